# baseline (speedup 1.0000x reference)
import functools

import jax
import jax.numpy as jnp
from jax import lax
from jax.experimental import pallas as pl
from jax.experimental.pallas import tpu as pltpu
from jax.experimental.pallas import tpu_sc as plsc

N = 50000
D = 256
K_OUT = 25000

R = 512
C = 128
NPAD = R * C

BLK = 8192
NBLK = 7

NW = 32
B_PER_W = 784
CHUNK = 112
NCHUNK = 7


def _score_body(x_ref, w_ref, b_ref, out_ref):
    out_ref[...] = (
        jnp.dot(x_ref[...], w_ref[...], preferred_element_type=jnp.float32)
        + b_ref[0, 0]
    )


def _scores(x, W, b):
    return pl.pallas_call(
        _score_body,
        grid=(NBLK,),
        in_specs=[
            pl.BlockSpec((BLK, D), lambda i: (i, 0)),
            pl.BlockSpec((D, 1), lambda i: (0, 0)),
            pl.BlockSpec(memory_space=pltpu.SMEM),
        ],
        out_specs=pl.BlockSpec((BLK, 1), lambda i: (i, 0)),
        out_shape=jax.ShapeDtypeStruct((NBLK * BLK, 1), jnp.float32),
    )(x, W, b.reshape(1, 1))


def _partner(a, m, axis, low):
    size = a.shape[axis]
    fwd = pltpu.roll(a, size - m, axis=axis)
    bwd = pltpu.roll(a, m, axis=axis)
    return jnp.where(low, fwd, bwd)


def _sort_body(s_ref, idx_ref):
    r_iota = lax.broadcasted_iota(jnp.int32, (R, C), 0)
    c_iota = lax.broadcasted_iota(jnp.int32, (R, C), 1)
    idx = c_iota * R + r_iota

    s = s_ref[...]
    bits = lax.bitcast_convert_type(s, jnp.int32)
    key = jnp.where(bits >= 0, bits, bits ^ jnp.int32(0x7FFFFFFF))
    key = jnp.where(s == 0.0, jnp.int32(0), key)
    key = jnp.where(idx >= N, jnp.int32(-(2**31)), key)

    kk = 2
    while kk <= NPAD:
        j = kk // 2
        while j >= 1:
            if j >= R:
                m, axis, pos = j // R, 1, c_iota
            else:
                m, axis, pos = j, 0, r_iota
            low = (pos & m) == 0
            if kk >= R:
                asc = (c_iota & (kk // R)) == 0
            else:
                asc = (r_iota & kk) == 0
            kp = _partner(key, m, axis, low)
            ip = _partner(idx, m, axis, low)
            cmp = (key > kp) | ((key == kp) & (idx < ip))
            take = jnp.logical_xor(cmp, low == asc)
            key = jnp.where(take, kp, key)
            idx = jnp.where(take, ip, idx)
            j //= 2
        kk *= 2

    idx_ref[...] = idx


def _sort(s_cm):
    return pl.pallas_call(
        _sort_body,
        out_shape=jax.ShapeDtypeStruct((R, C), jnp.int32),
    )(s_cm)


def _gather_body(x_hbm, idx_hbm, out_hbm, idx_v, rows_v, sem):
    wid = lax.axis_index("s") * 2 + lax.axis_index("c")
    base = wid * B_PER_W
    pltpu.sync_copy(idx_hbm.at[pl.ds(base, B_PER_W)], idx_v)
    for ch in range(NCHUNK):
        pltpu.async_copy(
            x_hbm.at[idx_v.at[pl.ds(ch * CHUNK, CHUNK)]], rows_v, sem
        ).wait()
        g0 = base + ch * CHUNK
        full = g0 + CHUNK <= K_OUT

        @pl.when(full)
        def _():
            pltpu.sync_copy(rows_v, out_hbm.at[pl.ds(g0, CHUNK)])

        @pl.when(jnp.logical_not(full))
        def _():
            tail = K_OUT - (NW * B_PER_W - CHUNK)
            pltpu.sync_copy(rows_v.at[pl.ds(0, tail)], out_hbm.at[pl.ds(g0, tail)])


def _gather(x, idx_pad):
    mesh = plsc.VectorSubcoreMesh(core_axis_name="c", subcore_axis_name="s")
    f = functools.partial(
        pl.kernel,
        mesh=mesh,
        out_type=jax.ShapeDtypeStruct((K_OUT, D), jnp.float32),
        scratch_types=[
            pltpu.VMEM((B_PER_W,), jnp.int32),
            pltpu.VMEM((CHUNK, D), jnp.float32),
            pltpu.SemaphoreType.DMA,
        ],
    )(_gather_body)
    return f(x, idx_pad)


def kernel(x, W, b):
    s = _scores(x, W, b)[:, 0]
    s = jnp.pad(s, (0, NPAD - NBLK * BLK))
    s_cm = s.reshape(C, R).T
    idx_grid = _sort(s_cm)
    idx_lin = idx_grid.T.reshape(-1)
    x_pool = _gather(x, idx_lin[: NW * B_PER_W])
    return (x_pool, idx_lin[:K_OUT])


# speedup vs baseline: 1.0410x; 1.0410x over previous
import functools

import jax
import jax.numpy as jnp
from jax import lax
from jax.experimental import pallas as pl
from jax.experimental.pallas import tpu as pltpu
from jax.experimental.pallas import tpu_sc as plsc

N = 50000
D = 256
K_OUT = 25000

R = 512
C = 128
NPAD = R * C

BLK = 8192
NBLK = 7
NROW = NBLK * BLK // C

INT_MIN = -(2**31)

NW = 32
B_PER_W = 784
CHUNK = 112
NCHUNK = 7


def _score_body(x_ref, w_ref, b_ref, out_ref):
    out_ref[...] = (
        jnp.dot(x_ref[...], w_ref[...], preferred_element_type=jnp.float32)
        + b_ref[0, 0]
    )


def _scores(x, W, b):
    return pl.pallas_call(
        _score_body,
        grid=(NBLK,),
        in_specs=[
            pl.BlockSpec((BLK, D), lambda i: (i, 0)),
            pl.BlockSpec((D, 1), lambda i: (0, 0)),
            pl.BlockSpec(memory_space=pltpu.SMEM),
        ],
        out_specs=pl.BlockSpec((BLK, 1), lambda i: (i, 0)),
        out_shape=jax.ShapeDtypeStruct((NBLK * BLK, 1), jnp.float32),
    )(x, W, b.reshape(1, 1))


def _partner(a, m, axis, low):
    size = a.shape[axis]
    fwd = pltpu.roll(a, size - m, axis=axis)
    bwd = pltpu.roll(a, m, axis=axis)
    return jnp.where(low, fwd, bwd)


def _stage(key, idx, r_io, c_io, kk, j):
    if j >= R:
        m, axis, pos = j // R, 1, c_io
    else:
        m, axis, pos = j, 0, r_io
    low = (pos & m) == 0
    if kk >= R:
        asc = (c_io & (kk // R)) == 0
    else:
        asc = (r_io & kk) == 0
    kp = _partner(key, m, axis, low)
    ip = _partner(idx, m, axis, low)
    cmp = (key > kp) | ((key == kp) & (idx < ip))
    take = jnp.logical_xor(cmp, low == asc)
    return jnp.where(take, kp, key), jnp.where(take, ip, idx)


def _sort_body(s_ref, out_ref):
    r_io = lax.broadcasted_iota(jnp.int32, (R, C), 0)
    c_io = lax.broadcasted_iota(jnp.int32, (R, C), 1)
    idx = r_io * C + c_io

    s = s_ref[...]
    bits = lax.bitcast_convert_type(s, jnp.int32)
    key = jnp.where(bits >= 0, bits, bits ^ jnp.int32(0x7FFFFFFF))
    key = jnp.where(s == 0.0, jnp.int32(0), key)
    key = jnp.where(idx[:NROW] >= N, jnp.int32(INT_MIN), key)
    key = jnp.concatenate([key, jnp.full((R - NROW, C), INT_MIN, jnp.int32)], 0)

    kk = 2
    while kk <= NPAD:
        j = kk // 2
        while j >= 1:
            key, idx = _stage(key, idx, r_io, c_io, kk, j)
            j //= 2
        kk *= 2

    out_ref[...] = jnp.transpose(idx[:, :64], (1, 0))


def _sort(s_rows):
    return pl.pallas_call(
        _sort_body,
        out_shape=jax.ShapeDtypeStruct((64, R), jnp.int32),
    )(s_rows)


def _gather_body(x_hbm, gidx_hbm, out_hbm, iout_hbm, idx_v, rows0, rows1, sem0, sem1):
    wid = lax.axis_index("s") * 2 + lax.axis_index("c")
    base = wid * B_PER_W
    pltpu.sync_copy(gidx_hbm.at[pl.ds(base, B_PER_W)], idx_v)
    bufs = (rows0, rows1)
    sems = (sem0, sem1)

    def start(ch):
        return pltpu.async_copy(
            x_hbm.at[idx_v.at[pl.ds(ch * CHUNK, CHUNK)]],
            bufs[ch % 2],
            sems[ch % 2],
        )

    cps = [None] * NCHUNK
    cps[0] = start(0)
    for ch in range(NCHUNK):
        if ch + 1 < NCHUNK:
            cps[ch + 1] = start(ch + 1)
        cps[ch].wait()
        g0 = base + ch * CHUNK
        full = g0 + CHUNK <= K_OUT

        @pl.when(full)
        def _():
            pltpu.sync_copy(bufs[ch % 2], out_hbm.at[pl.ds(g0, CHUNK)])
            pltpu.sync_copy(
                idx_v.at[pl.ds(ch * CHUNK, CHUNK)], iout_hbm.at[pl.ds(g0, CHUNK)]
            )

        @pl.when(jnp.logical_not(full))
        def _():
            tail = K_OUT - (NW * B_PER_W - CHUNK)
            pltpu.sync_copy(bufs[ch % 2].at[pl.ds(0, tail)], out_hbm.at[pl.ds(g0, tail)])
            pltpu.sync_copy(
                idx_v.at[pl.ds(ch * CHUNK, tail)], iout_hbm.at[pl.ds(g0, tail)]
            )


def _gather(x, gidx):
    mesh = plsc.VectorSubcoreMesh(core_axis_name="c", subcore_axis_name="s")
    f = functools.partial(
        pl.kernel,
        mesh=mesh,
        out_type=[
            jax.ShapeDtypeStruct((K_OUT, D), jnp.float32),
            jax.ShapeDtypeStruct((K_OUT,), jnp.int32),
        ],
        scratch_types=[
            pltpu.VMEM((B_PER_W,), jnp.int32),
            pltpu.VMEM((CHUNK, D), jnp.float32),
            pltpu.VMEM((CHUNK, D), jnp.float32),
            pltpu.SemaphoreType.DMA,
            pltpu.SemaphoreType.DMA,
        ],
    )(_gather_body)
    return f(x, gidx)


def kernel(x, W, b):
    s = _scores(x, W, b)
    rank_grid = _sort(s.reshape(NROW, C))
    gidx = rank_grid.reshape(-1)
    x_pool, idx = _gather(x, gidx)
    return (x_pool, idx)


# speedup vs baseline: 1.3148x; 1.2630x over previous
import functools

import jax
import jax.numpy as jnp
from jax import lax
from jax.experimental import pallas as pl
from jax.experimental.pallas import tpu as pltpu
from jax.experimental.pallas import tpu_sc as plsc

N = 50000
D = 256
K_OUT = 25000

R = 512
C = 128
NPAD = R * C

BLK = 8192
NBLK = 7
NROW = NBLK * BLK // C

INT_MIN = -(2**31)

NW = 32
B_PER_W = 784
CHUNK = 112
NCHUNK = 7


def _fused_body(x_ref, w_ref, b_ref, out_ref, ss_ref):
    step = pl.program_id(0)
    blk = (
        jnp.dot(x_ref[...], w_ref[...], preferred_element_type=jnp.float32)
        + b_ref[0, 0]
    )
    ss_ref[pl.ds(step * (BLK // C), BLK // C), :] = blk.reshape(BLK // C, C)

    @pl.when(step == NBLK - 1)
    def _():
        _sort_core(ss_ref[...], out_ref)


def _fused(x, W, b):
    return pl.pallas_call(
        _fused_body,
        grid=(NBLK,),
        in_specs=[
            pl.BlockSpec((BLK, D), lambda i: (i, 0)),
            pl.BlockSpec((D, 1), lambda i: (0, 0)),
            pl.BlockSpec(memory_space=pltpu.SMEM),
        ],
        out_specs=pl.BlockSpec((64, R), lambda i: (0, 0)),
        out_shape=jax.ShapeDtypeStruct((64, R), jnp.int32),
        scratch_shapes=[pltpu.VMEM((NROW, C), jnp.float32)],
    )(x, W, b.reshape(1, 1))


def _partner(a, m, axis, low):
    size = a.shape[axis]
    fwd = pltpu.roll(a, size - m, axis=axis)
    bwd = pltpu.roll(a, m, axis=axis)
    return jnp.where(low, fwd, bwd)


def _stage(key, idx, r_io, c_io, kk, j):
    if j >= R:
        m, axis, pos = j // R, 1, c_io
    else:
        m, axis, pos = j, 0, r_io
    low = (pos & m) == 0
    if kk >= R:
        asc = (c_io & (kk // R)) == 0
    else:
        asc = (r_io & kk) == 0
    kp = _partner(key, m, axis, low)
    ip = _partner(idx, m, axis, low)
    cmp = (key > kp) | ((key == kp) & (idx < ip))
    take = jnp.logical_xor(cmp, low == asc)
    return jnp.where(take, kp, key), jnp.where(take, ip, idx)


def _sort_core(s, out_ref):
    r_io = lax.broadcasted_iota(jnp.int32, (R, C), 0)
    c_io = lax.broadcasted_iota(jnp.int32, (R, C), 1)
    idx = r_io * C + c_io
    bits = lax.bitcast_convert_type(s, jnp.int32)
    key = jnp.where(bits >= 0, bits, bits ^ jnp.int32(0x7FFFFFFF))
    key = jnp.where(s == 0.0, jnp.int32(0), key)
    key = jnp.where(idx[:NROW] >= N, jnp.int32(INT_MIN), key)
    key = jnp.concatenate([key, jnp.full((R - NROW, C), INT_MIN, jnp.int32)], 0)

    kk = 2
    while kk <= NPAD:
        j = kk // 2
        while j >= 1:
            key, idx = _stage(key, idx, r_io, c_io, kk, j)
            j //= 2
        kk *= 2

    out_ref[...] = jnp.transpose(idx[:, :64], (1, 0))


def _gather_body(x_hbm, gidx_hbm, out_hbm, iout_hbm, idx_v, rows0, rows1, sem0, sem1):
    wid = lax.axis_index("s") * 2 + lax.axis_index("c")
    base = wid * B_PER_W
    pltpu.sync_copy(gidx_hbm.at[pl.ds(base, B_PER_W)], idx_v)
    bufs = (rows0, rows1)
    sems = (sem0, sem1)

    def start(ch):
        return pltpu.async_copy(
            x_hbm.at[idx_v.at[pl.ds(ch * CHUNK, CHUNK)]],
            bufs[ch % 2],
            sems[ch % 2],
        )

    cps = [None] * NCHUNK
    cps[0] = start(0)
    for ch in range(NCHUNK):
        if ch + 1 < NCHUNK:
            cps[ch + 1] = start(ch + 1)
        cps[ch].wait()
        g0 = base + ch * CHUNK
        full = g0 + CHUNK <= K_OUT

        @pl.when(full)
        def _():
            pltpu.sync_copy(bufs[ch % 2], out_hbm.at[pl.ds(g0, CHUNK)])
            pltpu.sync_copy(
                idx_v.at[pl.ds(ch * CHUNK, CHUNK)], iout_hbm.at[pl.ds(g0, CHUNK)]
            )

        @pl.when(jnp.logical_not(full))
        def _():
            tail = K_OUT - (NW * B_PER_W - CHUNK)
            pltpu.sync_copy(bufs[ch % 2].at[pl.ds(0, tail)], out_hbm.at[pl.ds(g0, tail)])
            pltpu.sync_copy(
                idx_v.at[pl.ds(ch * CHUNK, tail)], iout_hbm.at[pl.ds(g0, tail)]
            )


def _gather(x, gidx):
    mesh = plsc.VectorSubcoreMesh(core_axis_name="c", subcore_axis_name="s")
    f = functools.partial(
        pl.kernel,
        mesh=mesh,
        out_type=[
            jax.ShapeDtypeStruct((K_OUT, D), jnp.float32),
            jax.ShapeDtypeStruct((K_OUT,), jnp.int32),
        ],
        scratch_types=[
            pltpu.VMEM((B_PER_W,), jnp.int32),
            pltpu.VMEM((CHUNK, D), jnp.float32),
            pltpu.VMEM((CHUNK, D), jnp.float32),
            pltpu.SemaphoreType.DMA,
            pltpu.SemaphoreType.DMA,
        ],
    )(_gather_body)
    return f(x, gidx)


def kernel(x, W, b):
    rank_grid = _fused(x, W, b)
    gidx = rank_grid.reshape(-1)
    x_pool, idx = _gather(x, gidx)
    return (x_pool, idx)


# speedup vs baseline: 1.3233x; 1.0065x over previous
import functools

import jax
import jax.numpy as jnp
from jax import lax
from jax.experimental import pallas as pl
from jax.experimental.pallas import tpu as pltpu
from jax.experimental.pallas import tpu_sc as plsc

N = 50000
D = 256
K_OUT = 25000

R = 512
C = 128
NPAD = R * C

BLK = 8192
NBLK = 7
NROW = NBLK * BLK // C

INT_MIN = -(2**31)

NW = 32
B_PER_W = 784
CHUNK = 112
NCHUNK = 7


def _fused_body(x_ref, w_ref, b_ref, out_ref, key_scr, idx_scr):
    step = pl.program_id(0)
    nr = BLK // C
    blk = (
        jnp.dot(x_ref[...], w_ref[...], preferred_element_type=jnp.float32)
        + b_ref[0, 0]
    )
    s64 = blk.reshape(nr, C)
    r_abs = lax.broadcasted_iota(jnp.int32, (nr, C), 0) + step * nr
    c64 = lax.broadcasted_iota(jnp.int32, (nr, C), 1)
    node = r_abs * C + c64
    bits = lax.bitcast_convert_type(s64, jnp.int32)
    key = jnp.where(bits >= 0, bits, bits ^ jnp.int32(0x7FFFFFFF))
    key = jnp.where(s64 == 0.0, jnp.int32(0), key)
    key = jnp.where(node >= N, jnp.int32(INT_MIN), key)
    idx = node
    kk = 2
    while kk <= nr:
        j = kk // 2
        while j >= 1:
            key, idx = _stage(key, idx, r_abs, c64, kk, j)
            j //= 2
        kk *= 2
    key_scr[pl.ds(step * nr, nr), :] = key
    idx_scr[pl.ds(step * nr, nr), :] = idx

    @pl.when(step == NBLK - 1)
    def _():
        r_io = lax.broadcasted_iota(jnp.int32, (R, C), 0)
        c_io = lax.broadcasted_iota(jnp.int32, (R, C), 1)
        key = key_scr[...]
        idx = idx_scr[...]
        key = jnp.where(r_io >= NROW, jnp.int32(INT_MIN), key)
        idx = jnp.where(r_io >= NROW, r_io * C + c_io, idx)
        kk = 2 * nr
        while kk <= NPAD:
            j = kk // 2
            while j >= 1:
                if kk == NPAD and j == NPAD // 4:
                    key, idx = key[:, :64], idx[:, :64]
                    r_io, c_io = r_io[:, :64], c_io[:, :64]
                key, idx = _stage(key, idx, r_io, c_io, kk, j)
                j //= 2
            kk *= 2
        out_ref[...] = jnp.transpose(idx, (1, 0))


def _fused(x, W, b):
    return pl.pallas_call(
        _fused_body,
        grid=(NBLK,),
        in_specs=[
            pl.BlockSpec((BLK, D), lambda i: (i, 0)),
            pl.BlockSpec((D, 1), lambda i: (0, 0)),
            pl.BlockSpec(memory_space=pltpu.SMEM),
        ],
        out_specs=pl.BlockSpec((64, R), lambda i: (0, 0)),
        out_shape=jax.ShapeDtypeStruct((64, R), jnp.int32),
        scratch_shapes=[
            pltpu.VMEM((R, C), jnp.int32),
            pltpu.VMEM((R, C), jnp.int32),
        ],
    )(x, W, b.reshape(1, 1))


def _partner(a, m, axis, low):
    size = a.shape[axis]
    fwd = pltpu.roll(a, size - m, axis=axis)
    bwd = pltpu.roll(a, m, axis=axis)
    return jnp.where(low, fwd, bwd)


def _stage(key, idx, r_io, c_io, kk, j):
    if j >= R:
        m, axis, pos = j // R, 1, c_io
    else:
        m, axis, pos = j, 0, r_io
    low = (pos & m) == 0
    if kk >= R:
        asc = (c_io & (kk // R)) == 0
    else:
        asc = (r_io & kk) == 0
    kp = _partner(key, m, axis, low)
    ip = _partner(idx, m, axis, low)
    cmp = (key > kp) | ((key == kp) & (idx < ip))
    take = jnp.logical_xor(cmp, low == asc)
    return jnp.where(take, kp, key), jnp.where(take, ip, idx)


def _gather_body(x_hbm, gidx_hbm, out_hbm, iout_hbm, idx_v, rows0, rows1, sem0, sem1):
    wid = lax.axis_index("s") * 2 + lax.axis_index("c")
    base = wid * B_PER_W
    pltpu.sync_copy(gidx_hbm.at[pl.ds(base, B_PER_W)], idx_v)
    bufs = (rows0, rows1)
    sems = (sem0, sem1)

    def start(ch):
        return pltpu.async_copy(
            x_hbm.at[idx_v.at[pl.ds(ch * CHUNK, CHUNK)]],
            bufs[ch % 2],
            sems[ch % 2],
        )

    cps = [None] * NCHUNK
    cps[0] = start(0)
    for ch in range(NCHUNK):
        if ch + 1 < NCHUNK:
            cps[ch + 1] = start(ch + 1)
        cps[ch].wait()
        g0 = base + ch * CHUNK
        full = g0 + CHUNK <= K_OUT

        @pl.when(full)
        def _():
            pltpu.sync_copy(bufs[ch % 2], out_hbm.at[pl.ds(g0, CHUNK)])
            pltpu.sync_copy(
                idx_v.at[pl.ds(ch * CHUNK, CHUNK)], iout_hbm.at[pl.ds(g0, CHUNK)]
            )

        @pl.when(jnp.logical_not(full))
        def _():
            tail = K_OUT - (NW * B_PER_W - CHUNK)
            pltpu.sync_copy(bufs[ch % 2].at[pl.ds(0, tail)], out_hbm.at[pl.ds(g0, tail)])
            pltpu.sync_copy(
                idx_v.at[pl.ds(ch * CHUNK, tail)], iout_hbm.at[pl.ds(g0, tail)]
            )


def _gather(x, gidx):
    mesh = plsc.VectorSubcoreMesh(core_axis_name="c", subcore_axis_name="s")
    f = functools.partial(
        pl.kernel,
        mesh=mesh,
        out_type=[
            jax.ShapeDtypeStruct((K_OUT, D), jnp.float32),
            jax.ShapeDtypeStruct((K_OUT,), jnp.int32),
        ],
        scratch_types=[
            pltpu.VMEM((B_PER_W,), jnp.int32),
            pltpu.VMEM((CHUNK, D), jnp.float32),
            pltpu.VMEM((CHUNK, D), jnp.float32),
            pltpu.SemaphoreType.DMA,
            pltpu.SemaphoreType.DMA,
        ],
    )(_gather_body)
    return f(x, gidx)


def kernel(x, W, b):
    rank_grid = _fused(x, W, b)
    gidx = rank_grid.reshape(-1)
    x_pool, idx = _gather(x, gidx)
    return (x_pool, idx)


# speedup vs baseline: 1.3453x; 1.0167x over previous
import functools

import jax
import jax.numpy as jnp
from jax import lax
from jax.experimental import pallas as pl
from jax.experimental.pallas import tpu as pltpu
from jax.experimental.pallas import tpu_sc as plsc

N = 50000
D = 256
K_OUT = 25000

R = 512
C = 128
NPAD = R * C

BLK = 8192
NBLK = 7
NROW = NBLK * BLK // C

INT_MIN = -(2**31)

NW = 32
B_PER_W = 784
CHUNK = 112
NCHUNK = 7


def _fused_body(x_ref, w_ref, b_ref, out_ref, key_scr, idx_scr):
    step = pl.program_id(0)
    nr = BLK // C
    blk = (
        jnp.dot(x_ref[...], w_ref[...], preferred_element_type=jnp.float32)
        + b_ref[0, 0]
    )
    s64 = blk.reshape(nr, C)
    r_abs = lax.broadcasted_iota(jnp.int32, (nr, C), 0) + step * nr
    c64 = lax.broadcasted_iota(jnp.int32, (nr, C), 1)
    node = r_abs * C + c64
    bits = lax.bitcast_convert_type(s64, jnp.int32)
    key = jnp.where(bits >= 0, bits, bits ^ jnp.int32(0x7FFFFFFF))
    key = jnp.where(s64 == 0.0, jnp.int32(0), key)
    key = jnp.where(node >= N, jnp.int32(INT_MIN), key)
    idx = node
    kk = 2
    while kk <= nr:
        j = kk // 2
        while j >= 1:
            key, idx = _stage(key, idx, r_abs, c64, kk, j)
            j //= 2
        kk *= 2
    key_scr[pl.ds(step * nr, nr), :] = key
    idx_scr[pl.ds(step * nr, nr), :] = idx

    @pl.when(step == NBLK - 1)
    def _():
        r_io = lax.broadcasted_iota(jnp.int32, (R, C), 0)
        c_io = lax.broadcasted_iota(jnp.int32, (R, C), 1)
        key = key_scr[...]
        idx = idx_scr[...]
        key = jnp.where(r_io >= NROW, jnp.int32(INT_MIN), key)
        idx = jnp.where(r_io >= NROW, r_io * C + c_io, idx)
        kk = 2 * nr
        while kk <= NPAD:
            j = kk // 2
            while j >= 1:
                if kk == NPAD and j == NPAD // 4:
                    key, idx = key[:, :64], idx[:, :64]
                    r_io, c_io = r_io[:, :64], c_io[:, :64]
                key, idx = _stage(key, idx, r_io, c_io, kk, j)
                j //= 2
            kk *= 2
        out_ref[...] = jnp.transpose(idx, (1, 0))


def _fused(x, W, b):
    return pl.pallas_call(
        _fused_body,
        grid=(NBLK,),
        in_specs=[
            pl.BlockSpec((BLK, D), lambda i: (i, 0)),
            pl.BlockSpec((D, 1), lambda i: (0, 0)),
            pl.BlockSpec(memory_space=pltpu.SMEM),
        ],
        out_specs=pl.BlockSpec((64, R), lambda i: (0, 0)),
        out_shape=jax.ShapeDtypeStruct((64, R), jnp.int32),
        scratch_shapes=[
            pltpu.VMEM((R, C), jnp.int32),
            pltpu.VMEM((R, C), jnp.int32),
        ],
    )(x, W, b.reshape(1, 1))


def _partner(a, m, axis, low):
    size = a.shape[axis]
    fwd = pltpu.roll(a, size - m, axis=axis)
    bwd = pltpu.roll(a, m, axis=axis)
    return jnp.where(low, fwd, bwd)


def _stage(key, idx, r_io, c_io, kk, j):
    if j >= R:
        m, axis, pos = j // R, 1, c_io
    else:
        m, axis, pos = j, 0, r_io
    low = (pos & m) == 0
    if kk >= R:
        asc = (c_io & (kk // R)) == 0
    else:
        asc = (r_io & kk) == 0
    kp = _partner(key, m, axis, low)
    ip = _partner(idx, m, axis, low)
    cmp = (key > kp) | ((key == kp) & (idx < ip))
    take = jnp.logical_xor(cmp, low == asc)
    return jnp.where(take, kp, key), jnp.where(take, ip, idx)


NBUF = 4


def _gather_body(
    x_hbm, gidx_hbm, out_hbm, iout_hbm, idx_v, rows0, rows1, rows2, rows3,
    sem0, sem1, sem2, sem3,
):
    wid = lax.axis_index("s") * 2 + lax.axis_index("c")
    base = wid * B_PER_W
    pltpu.sync_copy(gidx_hbm.at[pl.ds(base, B_PER_W)], idx_v)
    bufs = (rows0, rows1, rows2, rows3)
    sems = (sem0, sem1, sem2, sem3)

    def start(ch):
        return pltpu.async_copy(
            x_hbm.at[idx_v.at[pl.ds(ch * CHUNK, CHUNK)]],
            bufs[ch % NBUF],
            sems[ch % NBUF],
        )

    cps = [None] * NCHUNK
    for ch in range(NBUF - 1):
        cps[ch] = start(ch)
    for ch in range(NCHUNK):
        if ch + NBUF - 1 < NCHUNK:
            cps[ch + NBUF - 1] = start(ch + NBUF - 1)
        cps[ch].wait()
        g0 = base + ch * CHUNK
        full = g0 + CHUNK <= K_OUT

        @pl.when(full)
        def _():
            pltpu.sync_copy(bufs[ch % NBUF], out_hbm.at[pl.ds(g0, CHUNK)])
            pltpu.sync_copy(
                idx_v.at[pl.ds(ch * CHUNK, CHUNK)], iout_hbm.at[pl.ds(g0, CHUNK)]
            )

        @pl.when(jnp.logical_not(full))
        def _():
            tail = K_OUT - (NW * B_PER_W - CHUNK)
            pltpu.sync_copy(bufs[ch % NBUF].at[pl.ds(0, tail)], out_hbm.at[pl.ds(g0, tail)])
            pltpu.sync_copy(
                idx_v.at[pl.ds(ch * CHUNK, tail)], iout_hbm.at[pl.ds(g0, tail)]
            )


def _gather(x, gidx):
    mesh = plsc.VectorSubcoreMesh(core_axis_name="c", subcore_axis_name="s")
    f = functools.partial(
        pl.kernel,
        mesh=mesh,
        out_type=[
            jax.ShapeDtypeStruct((K_OUT, D), jnp.float32),
            jax.ShapeDtypeStruct((K_OUT,), jnp.int32),
        ],
        scratch_types=[
            pltpu.VMEM((B_PER_W,), jnp.int32),
            pltpu.VMEM((CHUNK, D), jnp.float32),
            pltpu.VMEM((CHUNK, D), jnp.float32),
            pltpu.VMEM((CHUNK, D), jnp.float32),
            pltpu.VMEM((CHUNK, D), jnp.float32),
            pltpu.SemaphoreType.DMA,
            pltpu.SemaphoreType.DMA,
            pltpu.SemaphoreType.DMA,
            pltpu.SemaphoreType.DMA,
        ],
    )(_gather_body)
    return f(x, gidx)


def kernel(x, W, b):
    rank_grid = _fused(x, W, b)
    gidx = rank_grid.reshape(-1)
    x_pool, idx = _gather(x, gidx)
    return (x_pool, idx)


# speedup vs baseline: 1.4106x; 1.0485x over previous
import functools

import jax
import jax.numpy as jnp
from jax import lax
from jax.experimental import pallas as pl
from jax.experimental.pallas import tpu as pltpu
from jax.experimental.pallas import tpu_sc as plsc

N = 50000
D = 256
K_OUT = 25000

R = 512
C = 128
NPAD = R * C

BLK = 8192
NBLK = 7
NROW = NBLK * BLK // C

INT_MIN = -(2**31)

NW = 32
B_PER_W = 784
CHUNK = 112
NCHUNK = 7


def _fused_body(x_ref, w_ref, b_ref, out_ref, key_scr, idx_scr):
    step = pl.program_id(0)
    nr = BLK // C
    blk = (
        jnp.dot(x_ref[...], w_ref[...], preferred_element_type=jnp.float32)
        + b_ref[0, 0]
    )
    s64 = blk.reshape(nr, C)
    r_abs = lax.broadcasted_iota(jnp.int32, (nr, C), 0) + step * nr
    c64 = lax.broadcasted_iota(jnp.int32, (nr, C), 1)
    node = r_abs * C + c64
    bits = lax.bitcast_convert_type(s64, jnp.int32)
    key = jnp.where(bits >= 0, bits, bits ^ jnp.int32(0x7FFFFFFF))
    key = jnp.where(s64 == 0.0, jnp.int32(0), key)
    key = jnp.where(node >= N, jnp.int32(INT_MIN), key)
    idx = node
    kk = 2
    while kk <= nr:
        j = kk // 2
        while j >= 1:
            key, idx = _stage(key, idx, r_abs, c64, kk, j)
            j //= 2
        kk *= 2
    key_scr[pl.ds(step * nr, nr), :] = key
    idx_scr[pl.ds(step * nr, nr), :] = idx

    @pl.when(step == NBLK - 1)
    def _():
        r_io = lax.broadcasted_iota(jnp.int32, (R, C), 0)
        c_io = lax.broadcasted_iota(jnp.int32, (R, C), 1)
        key = key_scr[...]
        idx = idx_scr[...]
        key = jnp.where(r_io >= NROW, jnp.int32(INT_MIN), key)
        idx = jnp.where(r_io >= NROW, r_io * C + c_io, idx)
        kk = 2 * nr
        while kk <= NPAD:
            j = kk // 2
            while j >= 1:
                if kk == NPAD and j == NPAD // 4:
                    key, idx = key[:, :64], idx[:, :64]
                    r_io, c_io = r_io[:, :64], c_io[:, :64]
                key, idx = _stage(key, idx, r_io, c_io, kk, j)
                j //= 2
            kk *= 2
        out_ref[...] = jnp.transpose(idx, (1, 0))


def _fused(x, W, b):
    return pl.pallas_call(
        _fused_body,
        grid=(NBLK,),
        in_specs=[
            pl.BlockSpec((BLK, D), lambda i: (i, 0)),
            pl.BlockSpec((D, 1), lambda i: (0, 0)),
            pl.BlockSpec(memory_space=pltpu.SMEM),
        ],
        out_specs=pl.BlockSpec((64, R), lambda i: (0, 0)),
        out_shape=jax.ShapeDtypeStruct((64, R), jnp.int32),
        scratch_shapes=[
            pltpu.VMEM((R, C), jnp.int32),
            pltpu.VMEM((R, C), jnp.int32),
        ],
    )(x, W, b.reshape(1, 1))


def _partner(a, m, axis, low):
    size = a.shape[axis]
    fwd = pltpu.roll(a, size - m, axis=axis)
    bwd = pltpu.roll(a, m, axis=axis)
    return jnp.where(low, fwd, bwd)


def _stage_pairs(key, idx, r_io, c_io, kk, j):
    rc, w = key.shape
    g = rc // (2 * j)
    if kk >= R:
        asc = (c_io & (kk // R)) == 0
    else:
        asc = (r_io & kk) == 0
    a4 = lambda a: a.reshape(g, 2, j, w)
    k4, i4 = a4(key), a4(idx)
    klo, khi = k4[:, 0], k4[:, 1]
    ilo, ihi = i4[:, 0], i4[:, 1]
    asc_lo = a4(asc)[:, 0]
    cb = (klo > khi) | ((klo == khi) & (ilo < ihi))
    swap = jnp.logical_xor(cb, asc_lo)
    nkl = jnp.where(swap, khi, klo)
    nkh = jnp.where(swap, klo, khi)
    nil = jnp.where(swap, ihi, ilo)
    nih = jnp.where(swap, ilo, ihi)
    key = jnp.concatenate([nkl[:, None], nkh[:, None]], axis=1).reshape(rc, w)
    idx = jnp.concatenate([nil[:, None], nih[:, None]], axis=1).reshape(rc, w)
    return key, idx


def _stage(key, idx, r_io, c_io, kk, j):
    if 8 <= j < R:
        return _stage_pairs(key, idx, r_io, c_io, kk, j)
    if j >= R:
        m, axis, pos = j // R, 1, c_io
    else:
        m, axis, pos = j, 0, r_io
    low = (pos & m) == 0
    if kk >= R:
        asc = (c_io & (kk // R)) == 0
    else:
        asc = (r_io & kk) == 0
    kp = _partner(key, m, axis, low)
    ip = _partner(idx, m, axis, low)
    cmp = (key > kp) | ((key == kp) & (idx < ip))
    take = jnp.logical_xor(cmp, low == asc)
    return jnp.where(take, kp, key), jnp.where(take, ip, idx)


NBUF = 4


def _gather_body(
    x_hbm, gidx_hbm, out_hbm, iout_hbm, idx_v, rows0, rows1, rows2, rows3,
    sem0, sem1, sem2, sem3,
):
    wid = lax.axis_index("s") * 2 + lax.axis_index("c")
    base = wid * B_PER_W
    pltpu.sync_copy(gidx_hbm.at[pl.ds(base, B_PER_W)], idx_v)
    bufs = (rows0, rows1, rows2, rows3)
    sems = (sem0, sem1, sem2, sem3)

    def start(ch):
        return pltpu.async_copy(
            x_hbm.at[idx_v.at[pl.ds(ch * CHUNK, CHUNK)]],
            bufs[ch % NBUF],
            sems[ch % NBUF],
        )

    cps = [None] * NCHUNK
    for ch in range(NBUF - 1):
        cps[ch] = start(ch)
    for ch in range(NCHUNK):
        if ch + NBUF - 1 < NCHUNK:
            cps[ch + NBUF - 1] = start(ch + NBUF - 1)
        cps[ch].wait()
        g0 = base + ch * CHUNK
        full = g0 + CHUNK <= K_OUT

        @pl.when(full)
        def _():
            pltpu.sync_copy(bufs[ch % NBUF], out_hbm.at[pl.ds(g0, CHUNK)])
            pltpu.sync_copy(
                idx_v.at[pl.ds(ch * CHUNK, CHUNK)], iout_hbm.at[pl.ds(g0, CHUNK)]
            )

        @pl.when(jnp.logical_not(full))
        def _():
            tail = K_OUT - (NW * B_PER_W - CHUNK)
            pltpu.sync_copy(bufs[ch % NBUF].at[pl.ds(0, tail)], out_hbm.at[pl.ds(g0, tail)])
            pltpu.sync_copy(
                idx_v.at[pl.ds(ch * CHUNK, tail)], iout_hbm.at[pl.ds(g0, tail)]
            )


def _gather(x, gidx):
    mesh = plsc.VectorSubcoreMesh(core_axis_name="c", subcore_axis_name="s")
    f = functools.partial(
        pl.kernel,
        mesh=mesh,
        out_type=[
            jax.ShapeDtypeStruct((K_OUT, D), jnp.float32),
            jax.ShapeDtypeStruct((K_OUT,), jnp.int32),
        ],
        scratch_types=[
            pltpu.VMEM((B_PER_W,), jnp.int32),
            pltpu.VMEM((CHUNK, D), jnp.float32),
            pltpu.VMEM((CHUNK, D), jnp.float32),
            pltpu.VMEM((CHUNK, D), jnp.float32),
            pltpu.VMEM((CHUNK, D), jnp.float32),
            pltpu.SemaphoreType.DMA,
            pltpu.SemaphoreType.DMA,
            pltpu.SemaphoreType.DMA,
            pltpu.SemaphoreType.DMA,
        ],
    )(_gather_body)
    return f(x, gidx)


def kernel(x, W, b):
    rank_grid = _fused(x, W, b)
    gidx = rank_grid.reshape(-1)
    x_pool, idx = _gather(x, gidx)
    return (x_pool, idx)
